# baseline (speedup 1.0000x reference)
import functools

import jax
import jax.numpy as jnp
from jax.experimental import pallas as pl
from jax.experimental.pallas import tpu as pltpu


def _round_up(x, m):
    return ((x + m - 1) // m) * m


def _fused_convrnn_kernel(x_ref, w_ref, b_ref, m_ref, y_ref,
                          h1_ref, h2_ref, slab_ref, *,
                          T, cin, hd, kh, kw, W, NF, OFF, splits):
    ph, pw = kh // 2, kw // 2
    taps = [(OFF + (ki - ph) * W + (kj - pw), kj - pw)
            for ki in range(kh) for kj in range(kw)]
    KX = kh * kw * cin

    h1_ref[...] = jnp.zeros_like(h1_ref)
    h2_ref[...] = jnp.zeros_like(h2_ref)

    def put(row, n, src, dc):
        if dc == 0:
            slab_ref[row:row + n, :] = src
        else:
            slab_ref[row:row + n, :] = src * m_ref[dc + pw, :n, :]

    for k in range(T + 1):
        if k < T:
            for tap, (o, dc) in enumerate(taps):
                put(tap * cin, cin, x_ref[k, :, o:o + NF], dc)
        for tap, (o, dc) in enumerate(taps):
            put(KX + tap * hd, hd, h1_ref[:, o:o + NF], dc)
        for tap, (o, dc) in enumerate(taps):
            put(KX + (kh * kw + tap) * hd, hd, h2_ref[:, o:o + NF], dc)
        for s, nw in splits:
            acc = jnp.dot(w_ref[...], slab_ref[:, s:s + nw],
                          preferred_element_type=jnp.float32)
            g = jnp.tanh(acc + b_ref[...])
            if k < T:
                h1_ref[:, OFF + s:OFF + s + nw] = g[:hd].astype(h1_ref.dtype)
            if k >= 1:
                y_ref[k - 1, :, s:s + nw] = g[hd:]
                h2_ref[:, OFF + s:OFF + s + nw] = g[hd:].astype(h2_ref.dtype)


def _gate_slices(wx, wh, b, hd):
    wxg = wx[..., 3 * hd:4 * hd]
    whg = wh[..., 3 * hd:4 * hd]
    bg = b[:, 3 * hd:4 * hd].reshape(hd)
    wx2 = wxg.transpose(3, 0, 1, 2).reshape(hd, -1)
    wh2 = whg.transpose(3, 0, 1, 2).reshape(hd, -1)
    return wx2, wh2, bg


def kernel(x, wx0, wh0, b0, wx1, wh1, b1):
    T, B, cin, H, W = x.shape
    hd = wx0.shape[-1] // 4
    kh, kw = wx0.shape[0], wx0.shape[1]
    NF = H * W
    OFF = 128
    EXT = OFF + NF + OFF
    KX, KH = kh * kw * cin, kh * kw * hd
    K = KX + 2 * KH

    splits = (((0, NF // 2), (NF // 2, NF // 2)) if NF % 256 == 0
              else ((0, NF),))

    wx2_0, wh2_0, bg0 = _gate_slices(wx0, wh0, b0, hd)
    wx2_1, wh2_1, bg1 = _gate_slices(wx1, wh1, b1, hd)
    z_xh = jnp.zeros((hd, KX), jnp.float32)
    z_hh = jnp.zeros((hd, KH), jnp.float32)
    w_top = jnp.concatenate([wx2_0, wh2_0, z_hh], axis=1)
    w_bot = jnp.concatenate([z_xh, wx2_1, wh2_1], axis=1)
    w = jnp.concatenate([w_top, w_bot], axis=0).astype(jnp.bfloat16)
    bias = jnp.concatenate([bg0, bg1]).reshape(2 * hd, 1)

    xb = x.reshape(T, B, cin, NF)
    xb = jnp.pad(xb, ((0, 0), (0, 0), (0, 0), (OFF, OFF)))
    xb = xb.astype(jnp.bfloat16)

    ph, pw = kh // 2, kw // 2
    R = max(cin, hd)
    col = jnp.arange(NF) % W
    shifts = jnp.arange(-pw, pw + 1).reshape(-1, 1)
    cm = ((col[None, :] + shifts >= 0)
          & (col[None, :] + shifts < W)).astype(jnp.bfloat16)
    cm = jnp.broadcast_to(cm[:, None, :], (2 * pw + 1, R, NF))

    body = functools.partial(_fused_convrnn_kernel, T=T, cin=cin, hd=hd,
                             kh=kh, kw=kw, W=W, NF=NF, OFF=OFF,
                             splits=splits)

    y = pl.pallas_call(
        body,
        out_shape=jax.ShapeDtypeStruct((B, T, hd, NF), jnp.float32),
        grid=(B,),
        in_specs=[
            pl.BlockSpec((T, None, cin, EXT), lambda b: (0, b, 0, 0)),
            pl.BlockSpec((2 * hd, K), lambda b: (0, 0)),
            pl.BlockSpec((2 * hd, 1), lambda b: (0, 0)),
            pl.BlockSpec((2 * pw + 1, R, NF), lambda b: (0, 0, 0)),
        ],
        out_specs=pl.BlockSpec((None, T, hd, NF), lambda b: (b, 0, 0, 0)),
        scratch_shapes=[
            pltpu.VMEM((hd, EXT), jnp.bfloat16),
            pltpu.VMEM((hd, EXT), jnp.bfloat16),
            pltpu.VMEM((K, NF), jnp.bfloat16),
        ],
        compiler_params=pltpu.CompilerParams(
            dimension_semantics=("arbitrary",),
        ),
        name="fused_convrnn2",
    )(xb, w, bias, cm)

    return y.reshape(B, T, hd, H, W)


# speedup vs baseline: 1.0759x; 1.0759x over previous
import functools

import jax
import jax.numpy as jnp
from jax.experimental import pallas as pl
from jax.experimental.pallas import tpu as pltpu


def _round_up(x, m):
    return ((x + m - 1) // m) * m


def _fused_convrnn_kernel(x_ref, w_ref, b_ref, m_ref, y_ref,
                          xe_ref, h1_ref, h2_ref, slab_ref, *,
                          T, cin, hd, kh, kw, W, NF, OFF, splits):
    ph, pw = kh // 2, kw // 2
    taps = [(OFF + (ki - ph) * W + (kj - pw), kj - pw)
            for ki in range(kh) for kj in range(kw)]
    KX = kh * kw * cin

    xe_ref[...] = jnp.zeros_like(xe_ref)
    h1_ref[...] = jnp.zeros_like(h1_ref)
    h2_ref[...] = jnp.zeros_like(h2_ref)

    def put(row, n, src, dc):
        if dc == 0:
            slab_ref[row:row + n, :] = src
        else:
            slab_ref[row:row + n, :] = src * m_ref[dc + pw, :n, :]

    for k in range(T + 1):
        if k < T:
            xe_ref[:, OFF:OFF + NF] = x_ref[k].astype(xe_ref.dtype)
            for tap, (o, dc) in enumerate(taps):
                put(tap * cin, cin, xe_ref[:, o:o + NF], dc)
        for tap, (o, dc) in enumerate(taps):
            put(KX + tap * hd, hd, h1_ref[:, o:o + NF], dc)
        for tap, (o, dc) in enumerate(taps):
            put(KX + (kh * kw + tap) * hd, hd, h2_ref[:, o:o + NF], dc)
        for s, nw in splits:
            acc = jnp.dot(w_ref[...], slab_ref[:, s:s + nw],
                          preferred_element_type=jnp.float32)
            g = jnp.tanh(acc + b_ref[...])
            if k < T:
                h1_ref[:, OFF + s:OFF + s + nw] = g[:hd].astype(h1_ref.dtype)
            if k >= 1:
                y_ref[k - 1, :, s:s + nw] = g[hd:]
                h2_ref[:, OFF + s:OFF + s + nw] = g[hd:].astype(h2_ref.dtype)


def _gate_slices(wx, wh, b, hd):
    wxg = wx[..., 3 * hd:4 * hd]
    whg = wh[..., 3 * hd:4 * hd]
    bg = b[:, 3 * hd:4 * hd].reshape(hd)
    wx2 = wxg.transpose(3, 0, 1, 2).reshape(hd, -1)
    wh2 = whg.transpose(3, 0, 1, 2).reshape(hd, -1)
    return wx2, wh2, bg


def kernel(x, wx0, wh0, b0, wx1, wh1, b1):
    T, B, cin, H, W = x.shape
    hd = wx0.shape[-1] // 4
    kh, kw = wx0.shape[0], wx0.shape[1]
    NF = H * W
    OFF = 128
    EXT = OFF + NF + OFF
    KX, KH = kh * kw * cin, kh * kw * hd
    K = KX + 2 * KH

    splits = (((0, NF // 2), (NF // 2, NF // 2)) if NF % 256 == 0
              else ((0, NF),))

    wx2_0, wh2_0, bg0 = _gate_slices(wx0, wh0, b0, hd)
    wx2_1, wh2_1, bg1 = _gate_slices(wx1, wh1, b1, hd)
    z_xh = jnp.zeros((hd, KX), jnp.float32)
    z_hh = jnp.zeros((hd, KH), jnp.float32)
    w_top = jnp.concatenate([wx2_0, wh2_0, z_hh], axis=1)
    w_bot = jnp.concatenate([z_xh, wx2_1, wh2_1], axis=1)
    w = jnp.concatenate([w_top, w_bot], axis=0).astype(jnp.bfloat16)
    bias = jnp.concatenate([bg0, bg1]).reshape(2 * hd, 1)

    xb = x.reshape(T, B, cin, NF)

    ph, pw = kh // 2, kw // 2
    R = max(cin, hd)
    col = jnp.arange(NF) % W
    shifts = jnp.arange(-pw, pw + 1).reshape(-1, 1)
    cm = ((col[None, :] + shifts >= 0)
          & (col[None, :] + shifts < W)).astype(jnp.bfloat16)
    cm = jnp.broadcast_to(cm[:, None, :], (2 * pw + 1, R, NF))

    body = functools.partial(_fused_convrnn_kernel, T=T, cin=cin, hd=hd,
                             kh=kh, kw=kw, W=W, NF=NF, OFF=OFF,
                             splits=splits)

    y = pl.pallas_call(
        body,
        out_shape=jax.ShapeDtypeStruct((B, T, hd, NF), jnp.float32),
        grid=(B,),
        in_specs=[
            pl.BlockSpec((T, None, cin, NF), lambda b: (0, b, 0, 0)),
            pl.BlockSpec((2 * hd, K), lambda b: (0, 0)),
            pl.BlockSpec((2 * hd, 1), lambda b: (0, 0)),
            pl.BlockSpec((2 * pw + 1, R, NF), lambda b: (0, 0, 0)),
        ],
        out_specs=pl.BlockSpec((None, T, hd, NF), lambda b: (b, 0, 0, 0)),
        scratch_shapes=[
            pltpu.VMEM((cin, EXT), jnp.bfloat16),
            pltpu.VMEM((hd, EXT), jnp.bfloat16),
            pltpu.VMEM((hd, EXT), jnp.bfloat16),
            pltpu.VMEM((K, NF), jnp.bfloat16),
        ],
        compiler_params=pltpu.CompilerParams(
            dimension_semantics=("arbitrary",),
        ),
        name="fused_convrnn2",
    )(xb, w, bias, cm)

    return y.reshape(B, T, hd, H, W)


# speedup vs baseline: 2.1343x; 1.9837x over previous
import functools

import jax
import jax.numpy as jnp
from jax.experimental import pallas as pl
from jax.experimental.pallas import tpu as pltpu


def _fused_convrnn_kernel(x_ref, w_ref, b_ref, m_ref, y_ref,
                          xe_ref, h1_ref, h2_ref, slab_ref, *,
                          T, cin, hd, kh, kw, W, NF, OFF, splits):
    ph, pw = kh // 2, kw // 2
    rows = [OFF + dr * W for dr in range(-ph, ph + 1)]

    xe_ref[...] = jnp.zeros_like(xe_ref)
    h1_ref[...] = jnp.zeros_like(h1_ref)
    h2_ref[...] = jnp.zeros_like(h2_ref)

    for k in range(T + 1):
        if k < T:
            xe_ref[:, OFF:OFF + NF] = x_ref[k].astype(xe_ref.dtype)
            for i, o in enumerate(rows):
                slab_ref[i * cin:(i + 1) * cin, :] = xe_ref[:, o:o + NF]
        base = kh * cin
        for i, o in enumerate(rows):
            r = base + i * hd
            slab_ref[r:r + hd, :] = h1_ref[:, o:o + NF]
        base = kh * (cin + hd)
        for i, o in enumerate(rows):
            r = base + i * hd
            slab_ref[r:r + hd, :] = h2_ref[:, o:o + NF]
        for s, nw in splits:
            acc = jnp.dot(w_ref[pw], slab_ref[:, s:s + nw],
                          preferred_element_type=jnp.float32)
            for dc in range(-pw, pw + 1):
                if dc == 0:
                    continue
                c = jnp.dot(w_ref[dc + pw], slab_ref[:, s:s + nw],
                            preferred_element_type=jnp.float32)
                acc += (pltpu.roll(c, (-dc) % nw, axis=1)
                        * m_ref[dc + pw, :, s:s + nw])
            g = jnp.tanh(acc + b_ref[...])
            if k < T:
                h1_ref[:, OFF + s:OFF + s + nw] = g[:hd].astype(h1_ref.dtype)
            if k >= 1:
                y_ref[k - 1, :, s:s + nw] = g[hd:]
                h2_ref[:, OFF + s:OFF + s + nw] = g[hd:].astype(h2_ref.dtype)


def _gate_slices(wx, wh, b, hd):
    wxg = wx[..., 3 * hd:4 * hd]
    whg = wh[..., 3 * hd:4 * hd]
    bg = b[:, 3 * hd:4 * hd].reshape(hd)
    wx2 = wxg.transpose(1, 3, 0, 2).reshape(wx.shape[1], hd, -1)
    wh2 = whg.transpose(1, 3, 0, 2).reshape(wh.shape[1], hd, -1)
    return wx2, wh2, bg


def kernel(x, wx0, wh0, b0, wx1, wh1, b1):
    T, B, cin, H, W = x.shape
    hd = wx0.shape[-1] // 4
    kh, kw = wx0.shape[0], wx0.shape[1]
    ph, pw = kh // 2, kw // 2
    NF = H * W
    OFF = 128
    EXT = OFF + NF + OFF
    K = kh * (cin + 2 * hd)

    splits = (((0, NF // 2), (NF // 2, NF // 2))
              if (NF % 256 == 0 and (NF // 2) % W == 0) else ((0, NF),))

    wx2_0, wh2_0, bg0 = _gate_slices(wx0, wh0, b0, hd)
    wx2_1, wh2_1, bg1 = _gate_slices(wx1, wh1, b1, hd)
    z_xh = jnp.zeros((kw, hd, kh * cin), jnp.float32)
    z_hh = jnp.zeros((kw, hd, kh * hd), jnp.float32)
    w_top = jnp.concatenate([wx2_0, wh2_0, z_hh], axis=2)
    w_bot = jnp.concatenate([z_xh, wx2_1, wh2_1], axis=2)
    w = jnp.concatenate([w_top, w_bot], axis=1).astype(jnp.bfloat16)
    bias = jnp.concatenate([bg0, bg1]).reshape(2 * hd, 1)

    col = jnp.arange(NF) % W
    shifts = jnp.arange(-pw, pw + 1).reshape(-1, 1)
    cm = ((col[None, :] + shifts >= 0)
          & (col[None, :] + shifts < W)).astype(jnp.float32)
    cm = jnp.broadcast_to(cm[:, None, :], (kw, 2 * hd, NF))

    xb = x.reshape(T, B, cin, NF)

    body = functools.partial(_fused_convrnn_kernel, T=T, cin=cin, hd=hd,
                             kh=kh, kw=kw, W=W, NF=NF, OFF=OFF,
                             splits=splits)

    y = pl.pallas_call(
        body,
        out_shape=jax.ShapeDtypeStruct((B, T, hd, NF), jnp.float32),
        grid=(B,),
        in_specs=[
            pl.BlockSpec((T, None, cin, NF), lambda b: (0, b, 0, 0)),
            pl.BlockSpec((kw, 2 * hd, K), lambda b: (0, 0, 0)),
            pl.BlockSpec((2 * hd, 1), lambda b: (0, 0)),
            pl.BlockSpec((kw, 2 * hd, NF), lambda b: (0, 0, 0)),
        ],
        out_specs=pl.BlockSpec((None, T, hd, NF), lambda b: (b, 0, 0, 0)),
        scratch_shapes=[
            pltpu.VMEM((cin, EXT), jnp.bfloat16),
            pltpu.VMEM((hd, EXT), jnp.bfloat16),
            pltpu.VMEM((hd, EXT), jnp.bfloat16),
            pltpu.VMEM((K, NF), jnp.bfloat16),
        ],
        compiler_params=pltpu.CompilerParams(
            dimension_semantics=("arbitrary",),
        ),
        name="fused_convrnn2",
    )(xb, w, bias, cm)

    return y.reshape(B, T, hd, H, W)


# speedup vs baseline: 2.1760x; 1.0195x over previous
import functools

import jax
import jax.numpy as jnp
from jax.experimental import pallas as pl
from jax.experimental.pallas import tpu as pltpu


def _fused_convrnn_kernel(x_ref, w_ref, b_ref, m_ref, y_ref, slab_ref, *,
                          T, cin, hd, kh, kw, W, NF, OFF, splits):
    ph, pw = kh // 2, kw // 2
    drs = list(range(-ph, ph + 1))
    b1, b2 = kh * cin, kh * (cin + hd)

    slab_ref[...] = jnp.zeros_like(slab_ref)

    def put_bands(base, n, v, s, nw):
        for i, dr in enumerate(drs):
            o = OFF + s - dr * W
            slab_ref[base + i * n:base + (i + 1) * n, o:o + nw] = v

    for k in range(T + 1):
        if k < T:
            put_bands(0, cin, x_ref[k].astype(slab_ref.dtype), 0, NF)
        gs = []
        for s, nw in splits:
            acc = jnp.dot(w_ref[pw], slab_ref[:, OFF + s:OFF + s + nw],
                          preferred_element_type=jnp.float32)
            for dc in range(-pw, pw + 1):
                if dc == 0:
                    continue
                c = jnp.dot(w_ref[dc + pw], slab_ref[:, OFF + s:OFF + s + nw],
                            preferred_element_type=jnp.float32)
                acc += (pltpu.roll(c, (-dc) % nw, axis=1)
                        * m_ref[dc + pw, :, s:s + nw])
            gs.append(jnp.tanh(acc + b_ref[...]))
        for (s, nw), g in zip(splits, gs):
            if k < T:
                put_bands(b1, hd, g[:hd].astype(slab_ref.dtype), s, nw)
            if k >= 1:
                y_ref[k - 1, :, s:s + nw] = g[hd:]
                put_bands(b2, hd, g[hd:].astype(slab_ref.dtype), s, nw)


def _gate_slices(wx, wh, b, hd):
    wxg = wx[..., 3 * hd:4 * hd]
    whg = wh[..., 3 * hd:4 * hd]
    bg = b[:, 3 * hd:4 * hd].reshape(hd)
    wx2 = wxg.transpose(1, 3, 0, 2).reshape(wx.shape[1], hd, -1)
    wh2 = whg.transpose(1, 3, 0, 2).reshape(wh.shape[1], hd, -1)
    return wx2, wh2, bg


def kernel(x, wx0, wh0, b0, wx1, wh1, b1):
    T, B, cin, H, W = x.shape
    hd = wx0.shape[-1] // 4
    kh, kw = wx0.shape[0], wx0.shape[1]
    ph, pw = kh // 2, kw // 2
    NF = H * W
    OFF = 128
    EXT = OFF + NF + OFF
    K = kh * (cin + 2 * hd)

    splits = (((0, NF // 2), (NF // 2, NF // 2))
              if (NF % 256 == 0 and (NF // 2) % W == 0) else ((0, NF),))

    wx2_0, wh2_0, bg0 = _gate_slices(wx0, wh0, b0, hd)
    wx2_1, wh2_1, bg1 = _gate_slices(wx1, wh1, b1, hd)
    z_xh = jnp.zeros((kw, hd, kh * cin), jnp.float32)
    z_hh = jnp.zeros((kw, hd, kh * hd), jnp.float32)
    w_top = jnp.concatenate([wx2_0, wh2_0, z_hh], axis=2)
    w_bot = jnp.concatenate([z_xh, wx2_1, wh2_1], axis=2)
    w = jnp.concatenate([w_top, w_bot], axis=1).astype(jnp.bfloat16)
    bias = jnp.concatenate([bg0, bg1]).reshape(2 * hd, 1)

    col = jnp.arange(NF) % W
    shifts = jnp.arange(-pw, pw + 1).reshape(-1, 1)
    cm = ((col[None, :] + shifts >= 0)
          & (col[None, :] + shifts < W)).astype(jnp.float32)
    cm = jnp.broadcast_to(cm[:, None, :], (kw, 2 * hd, NF))

    xb = x.reshape(T, B, cin, NF)

    body = functools.partial(_fused_convrnn_kernel, T=T, cin=cin, hd=hd,
                             kh=kh, kw=kw, W=W, NF=NF, OFF=OFF,
                             splits=splits)

    y = pl.pallas_call(
        body,
        out_shape=jax.ShapeDtypeStruct((B, T, hd, NF), jnp.float32),
        grid=(B,),
        in_specs=[
            pl.BlockSpec((T, None, cin, NF), lambda b: (0, b, 0, 0)),
            pl.BlockSpec((kw, 2 * hd, K), lambda b: (0, 0, 0)),
            pl.BlockSpec((2 * hd, 1), lambda b: (0, 0)),
            pl.BlockSpec((kw, 2 * hd, NF), lambda b: (0, 0, 0)),
        ],
        out_specs=pl.BlockSpec((None, T, hd, NF), lambda b: (b, 0, 0, 0)),
        scratch_shapes=[
            pltpu.VMEM((K, EXT), jnp.bfloat16),
        ],
        compiler_params=pltpu.CompilerParams(
            dimension_semantics=("arbitrary",),
        ),
        name="fused_convrnn2",
    )(xb, w, bias, cm)

    return y.reshape(B, T, hd, H, W)
